# Initial kernel scaffold; baseline (speedup 1.0000x reference)
import jax
import jax.numpy as jnp
from jax.experimental import pallas as pl


def kernel(memory, last_update, nids, new_memory, new_last_update):
    raise NotImplementedError("write your pallas kernel here")


# speedup vs baseline: 42.6532x; 42.6532x over previous
import functools

import jax
import jax.numpy as jnp
from jax import lax
from jax.experimental import pallas as pl
from jax.experimental.pallas import tpu as pltpu
from jax.experimental.pallas import tpu_sc as plsc

_B = 16384
_D = 128
_CH = 128
_NCH = _B // _CH
_NW = 32
_RPW = _NCH // _NW
_EPW = _B // _NW
_N_TAG = 1 << 20

_mesh = plsc.VectorSubcoreMesh(core_axis_name="c", subcore_axis_name="s")


@functools.partial(
    pl.kernel,
    out_type=(
        jax.ShapeDtypeStruct((_B, _D), jnp.float32),
        jax.ShapeDtypeStruct((_B,), jnp.float32),
    ),
    mesh=_mesh,
    scratch_types=[
        pltpu.VMEM((_NCH, _CH), jnp.int32),
        pltpu.VMEM((_NCH, _CH), jnp.int32),
        pltpu.VMEM((_RPW, _CH), jnp.int32),
        pltpu.VMEM((_RPW, _CH), jnp.int32),
        pltpu.VMEM((_CH, _D), jnp.float32),
        pltpu.VMEM((_EPW,), jnp.float32),
        pltpu.VMEM_SHARED((_N_TAG,), jnp.int32),
    ],
)
def _sc_mem(nids2, jvals2, new_mem, new_lu, out_mem, out_lu,
            idx_full, jval_full, idx_w, w_w, rows_v, lu_v, tag_sh):
    c = lax.axis_index("c")
    s = lax.axis_index("s")
    wid = c * 16 + s

    @pl.when(s == 0)
    def _build_tag():
        pltpu.sync_copy(nids2, idx_full)
        pltpu.sync_copy(jvals2, jval_full)

        def body(q, carry):
            pltpu.sync_copy(jval_full.at[q], tag_sh.at[idx_full.at[q]])
            return carry

        lax.fori_loop(0, _NCH, body, 0)

    plsc.subcore_barrier()

    base_row = wid * _RPW
    pltpu.sync_copy(nids2.at[pl.ds(base_row, _RPW)], idx_w)
    for q in range(_RPW):
        pltpu.sync_copy(tag_sh.at[idx_w.at[q]], w_w.at[q])
    for q in range(_RPW):
        pltpu.sync_copy(new_mem.at[w_w.at[q]], rows_v)
        pltpu.sync_copy(rows_v, out_mem.at[pl.ds((base_row + q) * _CH, _CH)])
        pltpu.sync_copy(new_lu.at[w_w.at[q]], lu_v.at[pl.ds(q * _CH, _CH)])
    pltpu.sync_copy(lu_v, out_lu.at[pl.ds(wid * _EPW, _EPW)])


def kernel(memory, last_update, nids, new_memory, new_last_update):
    del memory, last_update
    nids2 = nids.reshape(_NCH, _CH)
    jvals2 = jnp.arange(_B, dtype=jnp.int32).reshape(_NCH, _CH)
    return _sc_mem(nids2, jvals2, new_memory, new_last_update)


# speedup vs baseline: 59.3629x; 1.3918x over previous
import functools

import jax
import jax.numpy as jnp
from jax import lax
from jax.experimental import pallas as pl
from jax.experimental.pallas import tpu as pltpu
from jax.experimental.pallas import tpu_sc as plsc

_B = 16384
_D = 128
_NW = 32
_EPW = _B // _NW
_RCH = 128
_NQ = _EPW // _RCH
_N_TAG = 1000448

_mesh = plsc.VectorSubcoreMesh(core_axis_name="c", subcore_axis_name="s")


@functools.partial(
    pl.kernel,
    out_type=(
        jax.ShapeDtypeStruct((_B, _D), jnp.float32),
        jax.ShapeDtypeStruct((_B,), jnp.float32),
    ),
    mesh=_mesh,
    scratch_types=[
        pltpu.VMEM((_B,), jnp.int32),
        pltpu.VMEM((_B,), jnp.int32),
        pltpu.VMEM((_EPW,), jnp.int32),
        pltpu.VMEM((2, _RCH, _D), jnp.float32),
        pltpu.VMEM((_EPW,), jnp.float32),
        pltpu.VMEM_SHARED((_N_TAG,), jnp.int32),
        pltpu.SemaphoreType.DMA,
        pltpu.SemaphoreType.DMA,
    ],
)
def _sc_mem(nids_h, jvals_h, new_mem, new_lu, out_mem, out_lu,
            idx_v, val_v, w_v, rows_v, lu_v, tag_sh, sem_a, sem_b):
    c = lax.axis_index("c")
    s = lax.axis_index("s")
    wid = c * 16 + s
    base = wid * _EPW

    @pl.when(s == 0)
    def _build_tag():
        pltpu.sync_copy(nids_h, idx_v)
        pltpu.sync_copy(jvals_h, val_v)
        pltpu.sync_copy(val_v, tag_sh.at[idx_v])

    @pl.when(s != 0)
    def _stage_slice():
        pltpu.sync_copy(nids_h.at[pl.ds(base, _EPW)],
                        idx_v.at[pl.ds(base, _EPW)])

    plsc.subcore_barrier()

    pltpu.sync_copy(tag_sh.at[idx_v.at[pl.ds(base, _EPW)]], w_v)

    sems = (sem_a, sem_b)
    pending = [None, None]
    for q in range(_NQ):
        buf = q % 2
        pending[buf] = pltpu.async_copy(
            new_mem.at[w_v.at[pl.ds(q * _RCH, _RCH)]], rows_v.at[buf],
            sems[buf])
        if q >= 1:
            prev = (q - 1) % 2
            pending[prev].wait()
            pltpu.sync_copy(rows_v.at[prev],
                            out_mem.at[pl.ds(base + (q - 1) * _RCH, _RCH)])
    last = (_NQ - 1) % 2
    pending[last].wait()
    pltpu.sync_copy(rows_v.at[last],
                    out_mem.at[pl.ds(base + (_NQ - 1) * _RCH, _RCH)])

    pltpu.sync_copy(new_lu.at[w_v], lu_v)
    pltpu.sync_copy(lu_v, out_lu.at[pl.ds(base, _EPW)])


def kernel(memory, last_update, nids, new_memory, new_last_update):
    del memory, last_update
    jvals = jnp.arange(_B, dtype=jnp.int32)
    return _sc_mem(nids, jvals, new_memory, new_last_update)
